# baseline (speedup 1.0000x reference)
import math

import jax
import jax.numpy as jnp
from jax import lax
from jax.experimental import pallas as pl
from jax.experimental.pallas import tpu as pltpu
from jax.experimental.pallas import tpu_sc as plsc

_VOCAB = 1000000
_D = 64
_B = 4096
_S = 200

_NC = 2
_NS = 16
_NW = _NC * _NS

_VPW = 31200
_TBLK = 200
_NBLK = _VPW // _TBLK

_BPW = _B // _NW
_SPB = 8
_NSB = _S // _SPB


def _pe_table():
    pos = jnp.arange(0, _S, dtype=jnp.float32)[:, None]
    ang = pos * jnp.exp(
        -jnp.arange(0, _D, 2, dtype=jnp.float32) * math.log(1000.0) / _D)
    pe = jnp.zeros((_S, _D), dtype=jnp.float32)
    pe = pe.at[:, 0::2].set(jnp.sin(ang))
    pe = pe.at[:, 1::2].set(jnp.cos(ang))
    return pe


def _wid():
    return lax.axis_index("s") * _NC + lax.axis_index("c")


def _transpose_body(wt, wrm, in_t, out_t, sem_i0, sem_i1, sem_o0, sem_o1):
    sem_i = (sem_i0, sem_i1)
    sem_o = (sem_o0, sem_o1)
    w = _wid()
    v0 = w * _VPW
    nblk = jnp.where(w == _NW - 1, _NBLK + 8, _NBLK)
    iota = lax.iota(jnp.int32, 16)
    idxd = [iota + 16 * k for k in range(_D // 16)]

    def fire_in(slot, t):
        pltpu.async_copy(wt.at[pl.ds(0, _D), pl.ds(v0 + t * _TBLK, _TBLK)],
                         in_t.at[slot], sem_i[slot])

    def wait_in(slot):
        pltpu.make_async_copy(wt.at[pl.ds(0, _D), pl.ds(v0, _TBLK)],
                              in_t.at[slot], sem_i[slot]).wait()

    def fire_out(slot, t):
        pltpu.async_copy(out_t.at[slot],
                         wrm.at[pl.ds(v0 + t * _TBLK, _TBLK)], sem_o[slot])

    def wait_out(slot):
        pltpu.make_async_copy(out_t.at[slot],
                              wrm.at[pl.ds(v0, _TBLK)], sem_o[slot]).wait()

    def transpose_block(slot):
        @pl.loop(0, _TBLK)
        def _(v):
            vv = jnp.full((16,), v, jnp.int32)
            for k in range(_D // 16):
                vals = plsc.load_gather(in_t.at[slot], [idxd[k], vv])
                out_t[slot, v, pl.ds(16 * k, 16)] = vals

    fire_in(0, 0)
    fire_in(1, 1)

    @pl.loop(0, nblk, step=2)
    def _(c):
        for slot in range(2):
            t = c + slot
            wait_in(slot)

            @pl.when(t >= 2)
            def _():
                wait_out(slot)

            transpose_block(slot)
            fire_out(slot, t)

            @pl.when(t + 2 < nblk)
            def _():
                fire_in(slot, t + 2)

    wait_out(0)
    wait_out(1)


def _gather_body(wrm, xt, pe, out, idx_v, gbuf, tbuf, pe_sb,
                 sem_i0, sem_i1, sem_g0, sem_g1, sem_o0, sem_o1,
                 sem_p0, sem_p1):
    sem_i = (sem_i0, sem_i1)
    sem_g = (sem_g0, sem_g1)
    sem_o = (sem_o0, sem_o1)
    sem_p = (sem_p0, sem_p1)
    w = _wid()
    bt0 = w * _BPW
    iota = lax.iota(jnp.int32, 16)
    idxb = [iota + 16 * m for m in range(_BPW // 16)]

    def fire_idx(q, sb):
        pltpu.async_copy(xt.at[pl.ds(sb * _SPB, _SPB), pl.ds(bt0, _BPW)],
                         idx_v.at[q], sem_i[q])

    def wait_idx(q):
        pltpu.make_async_copy(xt.at[pl.ds(0, _SPB), pl.ds(bt0, _BPW)],
                              idx_v.at[q], sem_i[q]).wait()

    def fire_gather(g, q, r, s):
        pltpu.async_copy(wrm.at[idx_v.at[q, r]], gbuf.at[g], sem_g[g])
        pltpu.async_copy(pe.at[s], pe_sb.at[g], sem_p[g])

    def wait_gather(g, q, r):
        pltpu.make_async_copy(wrm.at[idx_v.at[q, r]], gbuf.at[g],
                              sem_g[g]).wait()
        pltpu.make_async_copy(pe.at[0], pe_sb.at[g], sem_p[g]).wait()

    def fire_store(g, s):
        pltpu.async_copy(tbuf.at[g], out.at[s, pl.ds(0, 8), w], sem_o[g])

    def wait_store(g):
        pltpu.make_async_copy(tbuf.at[g], out.at[0, pl.ds(0, 8), w],
                              sem_o[g]).wait()

    def transpose_add(g):
        @pl.loop(0, _D)
        def _(d):
            dt = d >> 3
            r = d & 7
            pev = pe_sb[g, d, pl.ds(0, 16)]
            dd = jnp.full((16,), d, jnp.int32)
            for m in range(_BPW // 16):
                vals = plsc.load_gather(gbuf.at[g], [idxb[m], dd])
                tbuf[g, dt, r, pl.ds(16 * m, 16)] = vals + pev

    fire_idx(0, 0)
    fire_idx(1, 1)
    wait_idx(0)
    fire_gather(0, 0, 0, 0)

    @pl.loop(0, _NSB - 1, step=2)
    def _(c):
        for i in range(16):
            s = 8 * c + i
            g = i & 1
            q = (i >> 3) & 1
            r = i & 7
            wait_gather(g, q, r)
            if i == 7:
                wait_idx(1)
                fire_gather(g ^ 1, 1, 0, s + 1)
                @pl.when(c + 2 < _NSB)
                def _():
                    fire_idx(0, c + 2)
            elif i == 15:
                @pl.when(c + 2 < _NSB)
                def _():
                    wait_idx(0)
                    fire_gather(g ^ 1, 0, 0, s + 1)
                    @pl.when(c + 3 < _NSB)
                    def _():
                        fire_idx(1, c + 3)
            else:
                fire_gather(g ^ 1, ((i + 1) >> 3) & 1, (i + 1) & 7, s + 1)
            if i >= 2:
                wait_store(g)
            else:
                @pl.when(c > 0)
                def _():
                    wait_store(g)
            transpose_add(g)
            fire_store(g, s)

    for i in range(8):
        s = _S - 8 + i
        g = i & 1
        wait_gather(g, 0, i)
        if i < 7:
            fire_gather(g ^ 1, 0, i + 1, s + 1)
        wait_store(g)
        transpose_add(g)
        fire_store(g, s)

    wait_store(0)
    wait_store(1)


def kernel(x, W):
    pe = jnp.broadcast_to(_pe_table()[:, :, None], (_S, _D, 16)) + 0.0
    wt = W.T
    xt = x.astype(jnp.int32).T
    mesh = plsc.VectorSubcoreMesh(core_axis_name="c", subcore_axis_name="s")
    params = pltpu.CompilerParams(use_tc_tiling_on_sc=False, needs_layout_passes=False)

    wrm = pl.kernel(
        _transpose_body,
        out_type=jax.ShapeDtypeStruct((_VOCAB, _D), jnp.float32),
        mesh=mesh,
        compiler_params=params,
        scratch_types=[
            pltpu.VMEM((2, _D, _TBLK), jnp.float32),
            pltpu.VMEM((2, _TBLK, _D), jnp.float32),
            pltpu.SemaphoreType.DMA,
            pltpu.SemaphoreType.DMA,
            pltpu.SemaphoreType.DMA,
            pltpu.SemaphoreType.DMA,
        ],
    )(wt)

    L = pl.kernel(
        _gather_body,
        out_type=jax.ShapeDtypeStruct((_S, 8, _NW, 8, 128), jnp.float32),
        mesh=mesh,
        compiler_params=params,
        scratch_types=[
            pltpu.VMEM((2, _SPB, _BPW), jnp.int32),
            pltpu.VMEM((2, _BPW, _D), jnp.float32),
            pltpu.VMEM((2, 8, 8, 128), jnp.float32),
            pltpu.VMEM((2, _D, 16), jnp.float32),
            pltpu.SemaphoreType.DMA,
            pltpu.SemaphoreType.DMA,
            pltpu.SemaphoreType.DMA,
            pltpu.SemaphoreType.DMA,
            pltpu.SemaphoreType.DMA,
            pltpu.SemaphoreType.DMA,
            pltpu.SemaphoreType.DMA,
            pltpu.SemaphoreType.DMA,
        ],
    )(wrm, xt, pe)

    return L.transpose(2, 4, 0, 1, 3).reshape(_B, _S, _D)


# speedup vs baseline: 1.1306x; 1.1306x over previous
import math

import jax
import jax.numpy as jnp
from jax import lax
from jax.experimental import pallas as pl
from jax.experimental.pallas import tpu as pltpu
from jax.experimental.pallas import tpu_sc as plsc

_VOCAB = 1000000
_D = 64
_B = 4096
_S = 200

_NC = 2
_NS = 16
_NW = _NC * _NS

_VPW = 31200
_TBLK = 200
_NBLK = _VPW // _TBLK

_BPW = _B // _NW
_SPB = 8
_NSB = _S // _SPB


def _pe_table():
    pos = jnp.arange(0, _S, dtype=jnp.float32)[:, None]
    ang = pos * jnp.exp(
        -jnp.arange(0, _D, 2, dtype=jnp.float32) * math.log(1000.0) / _D)
    pe = jnp.zeros((_S, _D), dtype=jnp.float32)
    pe = pe.at[:, 0::2].set(jnp.sin(ang))
    pe = pe.at[:, 1::2].set(jnp.cos(ang))
    return pe


def _wid():
    return lax.axis_index("s") * _NC + lax.axis_index("c")


def _transpose_body(wt, wrm, in_t, out_t, sem_i0, sem_i1, sem_o0, sem_o1):
    sem_i = (sem_i0, sem_i1)
    sem_o = (sem_o0, sem_o1)
    w = _wid()
    v0 = w * _VPW
    nblk = jnp.where(w == _NW - 1, _NBLK + 8, _NBLK)
    iota = lax.iota(jnp.int32, 16)
    idxv = [iota + 16 * k for k in range(13)]
    tailmask = iota < 8

    def fire_in(slot, t):
        pltpu.async_copy(wt.at[pl.ds(0, _D), pl.ds(v0 + t * _TBLK, _TBLK)],
                         in_t.at[slot, pl.ds(0, _D), pl.ds(0, _TBLK)],
                         sem_i[slot])

    def wait_in(slot):
        pltpu.make_async_copy(wt.at[pl.ds(0, _D), pl.ds(v0, _TBLK)],
                              in_t.at[slot, pl.ds(0, _D), pl.ds(0, _TBLK)],
                              sem_i[slot]).wait()

    def fire_out(slot, t):
        pltpu.async_copy(out_t.at[slot, pl.ds(0, _TBLK), pl.ds(0, _D)],
                         wrm.at[pl.ds(v0 + t * _TBLK, _TBLK)], sem_o[slot])

    def wait_out(slot):
        pltpu.make_async_copy(out_t.at[slot, pl.ds(0, _TBLK), pl.ds(0, _D)],
                              wrm.at[pl.ds(v0, _TBLK)], sem_o[slot]).wait()

    def transpose_block(slot):
        @pl.loop(0, _D)
        def _(d):
            dd = jnp.full((16,), d, jnp.int32)
            for k in range(13):
                vals = in_t[slot, d, pl.ds(16 * k, 16)]
                if k < 12:
                    plsc.store_scatter(out_t.at[slot], [idxv[k], dd], vals)
                else:
                    plsc.store_scatter(out_t.at[slot], [idxv[k], dd], vals,
                                       mask=tailmask)

    fire_in(0, 0)
    fire_in(1, 1)

    @pl.loop(0, nblk, step=2)
    def _(c):
        for slot in range(2):
            t = c + slot
            wait_in(slot)

            @pl.when(t >= 2)
            def _():
                wait_out(slot)

            transpose_block(slot)
            fire_out(slot, t)

            @pl.when(t + 2 < nblk)
            def _():
                fire_in(slot, t + 2)

    wait_out(0)
    wait_out(1)


def _gather_body(wrm, xt, pe, out, idx_v, gbuf, tbuf, pe_sb,
                 sem_i0, sem_i1, sem_g0, sem_g1, sem_o0, sem_o1,
                 sem_p0, sem_p1):
    sem_i = (sem_i0, sem_i1)
    sem_g = (sem_g0, sem_g1)
    sem_o = (sem_o0, sem_o1)
    sem_p = (sem_p0, sem_p1)
    w = _wid()
    bt0 = w * _BPW
    iota = lax.iota(jnp.int32, 16)
    idxdt = [(iota + 16 * h) >> 3 for h in range(_D // 16)]
    idxr = [(iota + 16 * h) & 7 for h in range(_D // 16)]

    def fire_idx(q, sb):
        pltpu.async_copy(xt.at[pl.ds(sb * _SPB, _SPB), pl.ds(bt0, _BPW)],
                         idx_v.at[q], sem_i[q])

    def wait_idx(q):
        pltpu.make_async_copy(xt.at[pl.ds(0, _SPB), pl.ds(bt0, _BPW)],
                              idx_v.at[q], sem_i[q]).wait()

    def fire_gather(g, q, r, s):
        pltpu.async_copy(wrm.at[idx_v.at[q, r]], gbuf.at[g], sem_g[g])
        pltpu.async_copy(pe.at[s], pe_sb.at[g], sem_p[g])

    def wait_gather(g, q, r):
        pltpu.make_async_copy(wrm.at[idx_v.at[q, r]], gbuf.at[g],
                              sem_g[g]).wait()
        pltpu.make_async_copy(pe.at[0], pe_sb.at[g], sem_p[g]).wait()

    def fire_store(g, s):
        pltpu.async_copy(
            tbuf.at[g, pl.ds(0, 8), pl.ds(0, 8), pl.ds(0, 128)],
            out.at[s, pl.ds(0, 8), w], sem_o[g])

    def wait_store(g):
        pltpu.make_async_copy(
            tbuf.at[g, pl.ds(0, 8), pl.ds(0, 8), pl.ds(0, 128)],
            out.at[0, pl.ds(0, 8), w], sem_o[g]).wait()

    def transpose_add(g):
        pev = [pe_sb[g, pl.ds(16 * h, 16)] for h in range(_D // 16)]

        @pl.loop(0, _BPW, unroll=2)
        def _(bl):
            bb = jnp.full((16,), bl, jnp.int32)
            for h in range(_D // 16):
                vals = gbuf[g, bl, pl.ds(16 * h, 16)] + pev[h]
                plsc.store_scatter(tbuf.at[g], [idxdt[h], idxr[h], bb], vals)

    fire_idx(0, 0)
    fire_idx(1, 1)
    wait_idx(0)
    fire_gather(0, 0, 0, 0)

    @pl.loop(0, _NSB - 1, step=2)
    def _(c):
        for i in range(16):
            s = 8 * c + i
            g = i & 1
            q = (i >> 3) & 1
            r = i & 7
            wait_gather(g, q, r)
            if i == 7:
                wait_idx(1)
                fire_gather(g ^ 1, 1, 0, s + 1)
                @pl.when(c + 2 < _NSB)
                def _():
                    fire_idx(0, c + 2)
            elif i == 15:
                @pl.when(c + 2 < _NSB)
                def _():
                    wait_idx(0)
                    fire_gather(g ^ 1, 0, 0, s + 1)
                    @pl.when(c + 3 < _NSB)
                    def _():
                        fire_idx(1, c + 3)
            else:
                fire_gather(g ^ 1, ((i + 1) >> 3) & 1, (i + 1) & 7, s + 1)
            if i >= 2:
                wait_store(g)
            else:
                @pl.when(c > 0)
                def _():
                    wait_store(g)
            transpose_add(g)
            fire_store(g, s)

    for i in range(8):
        s = _S - 8 + i
        g = i & 1
        wait_gather(g, 0, i)
        if i < 7:
            fire_gather(g ^ 1, 0, i + 1, s + 1)
        wait_store(g)
        transpose_add(g)
        fire_store(g, s)

    wait_store(0)
    wait_store(1)


def kernel(x, W):
    pe = _pe_table()
    wt = W.T
    xt = x.astype(jnp.int32).T
    mesh = plsc.VectorSubcoreMesh(core_axis_name="c", subcore_axis_name="s")
    params = pltpu.CompilerParams(use_tc_tiling_on_sc=False, needs_layout_passes=False)

    wrm = pl.kernel(
        _transpose_body,
        out_type=jax.ShapeDtypeStruct((_VOCAB, _D), jnp.float32),
        mesh=mesh,
        compiler_params=params,
        scratch_types=[
            pltpu.VMEM((2, _D, 208), jnp.float32),
            pltpu.VMEM((2, _TBLK, 65), jnp.float32),
            pltpu.SemaphoreType.DMA,
            pltpu.SemaphoreType.DMA,
            pltpu.SemaphoreType.DMA,
            pltpu.SemaphoreType.DMA,
        ],
    )(wt)

    L = pl.kernel(
        _gather_body,
        out_type=jax.ShapeDtypeStruct((_S, 8, _NW, 8, 128), jnp.float32),
        mesh=mesh,
        compiler_params=params,
        scratch_types=[
            pltpu.VMEM((2, _SPB, _BPW), jnp.int32),
            pltpu.VMEM((2, _BPW, _D), jnp.float32),
            pltpu.VMEM((2, 8, 8, 129), jnp.float32),
            pltpu.VMEM((2, _D), jnp.float32),
            pltpu.SemaphoreType.DMA,
            pltpu.SemaphoreType.DMA,
            pltpu.SemaphoreType.DMA,
            pltpu.SemaphoreType.DMA,
            pltpu.SemaphoreType.DMA,
            pltpu.SemaphoreType.DMA,
            pltpu.SemaphoreType.DMA,
            pltpu.SemaphoreType.DMA,
        ],
    )(wrm, xt, pe)

    return L.transpose(2, 4, 0, 1, 3).reshape(_B, _S, _D)


# speedup vs baseline: 5.4091x; 4.7843x over previous
import math

import jax
import jax.numpy as jnp
from jax import lax
from jax.experimental import pallas as pl
from jax.experimental.pallas import tpu as pltpu
from jax.experimental.pallas import tpu_sc as plsc

_VOCAB = 1000000
_D = 64
_B = 4096
_S = 200

_NC = 2
_NS = 16
_NW = _NC * _NS

_ROWS = _B * _S
_RPW = _ROWS // _NW
_CH = _S
_NCH = _RPW // _CH
_HALF = _CH // 2
_XROWS = _ROWS // _HALF


def _pe_table():
    pos = jnp.arange(0, _S, dtype=jnp.float32)[:, None]
    ang = pos * jnp.exp(
        -jnp.arange(0, _D, 2, dtype=jnp.float32) * math.log(1000.0) / _D)
    pe = jnp.zeros((_S, _D), dtype=jnp.float32)
    pe = pe.at[:, 0::2].set(jnp.sin(ang))
    pe = pe.at[:, 1::2].set(jnp.cos(ang))
    return pe.reshape(2, _HALF, _D)


def _body(x2, W, pe, out, pe_v, idx_v, gbuf, obuf,
          sem_g0, sem_g1, sem_o0, sem_o1):
    sem_g = (sem_g0, sem_g1)
    sem_o = (sem_o0, sem_o1)
    cid = lax.axis_index("c")
    sid = lax.axis_index("s")
    wid = sid * _NC + cid
    xbase = wid * (_RPW // _HALF)

    pltpu.sync_copy(pe, pe_v)

    def fire_gathers(b, cc):
        r0 = xbase + cc * 2
        pltpu.sync_copy(x2.at[pl.ds(r0, 2)], idx_v.at[b])
        for j in range(2):
            pltpu.async_copy(W.at[idx_v.at[b, j]], gbuf.at[b, j], sem_g[b])

    def wait_gathers(b):
        for j in range(2):
            pltpu.make_async_copy(W.at[idx_v.at[b, j]], gbuf.at[b, j],
                                  sem_g[b]).wait()

    def fire_store(b, cc):
        r0 = xbase + cc * 2
        pltpu.async_copy(obuf.at[b], out.at[pl.ds(r0, 2)], sem_o[b])

    def wait_store(b):
        pltpu.make_async_copy(obuf.at[b], out.at[pl.ds(0, 2)], sem_o[b]).wait()

    def add_pe(b):
        @pl.loop(0, _HALF)
        def _(r):
            for j in range(2):
                for k in range(_D // 16):
                    sl = pl.ds(k * 16, 16)
                    obuf[b, j, r, sl] = gbuf[b, j, r, sl] + pe_v[j, r, sl]

    fire_gathers(0, 0)
    fire_gathers(1, 1)

    @pl.loop(0, _NCH, step=2)
    def _(c):
        for b in range(2):
            cc = c + b
            wait_gathers(b)

            @pl.when(cc >= 2)
            def _():
                wait_store(b)

            add_pe(b)

            @pl.when(cc + 2 < _NCH)
            def _():
                fire_gathers(b, cc + 2)

            fire_store(b, cc)

    wait_store(0)
    wait_store(1)


def kernel(x, W):
    pe = _pe_table()
    x2 = x.astype(jnp.int32).reshape(_XROWS, _HALF)
    call = pl.kernel(
        _body,
        out_type=jax.ShapeDtypeStruct((_XROWS, _HALF, _D), jnp.float32),
        mesh=plsc.VectorSubcoreMesh(core_axis_name="c", subcore_axis_name="s"),
        compiler_params=pltpu.CompilerParams(use_tc_tiling_on_sc=False),
        scratch_types=[
            pltpu.VMEM((2, _HALF, _D), jnp.float32),
            pltpu.VMEM((2, 2, _HALF), jnp.int32),
            pltpu.VMEM((2, 2, _HALF, _D), jnp.float32),
            pltpu.VMEM((2, 2, _HALF, _D), jnp.float32),
            pltpu.SemaphoreType.DMA,
            pltpu.SemaphoreType.DMA,
            pltpu.SemaphoreType.DMA,
            pltpu.SemaphoreType.DMA,
        ],
    )
    out = call(x2, W, pe)
    return out.reshape(_B, _S, _D)
